# Initial kernel scaffold; baseline (speedup 1.0000x reference)
import jax
import jax.numpy as jnp
from jax.experimental import pallas as pl


def kernel(elem_weights, elem_in_fea, self_fea_idx, nbr_fea_idx, gate_w0, gate_b0, gate_w1, gate_b1, msg_w0, msg_b0, msg_w1, msg_b1, pow_p):
    raise NotImplementedError("write your pallas kernel here")


# speedup vs baseline: 7.8706x; 7.8706x over previous
import functools

import jax
import jax.numpy as jnp
from jax import lax
from jax.experimental import pallas as pl
from jax.experimental.pallas import tpu as pltpu
from jax.experimental.pallas import tpu_sc as plsc

N = 10000
M = 320000
D = 128
H = 3
HID = 256

NC = 2
NS = 16
NW = NC * NS

EPW = M // NW
GCH = 80
SCH = 80
NPAD = 10240
APT = NPAD // NS
DNR = NPAD // 8
DPT = DNR // NS
CW = H * D
VW = 16
F1 = 2 * H * HID
EB = 512


def _gather_sc(x, w, sidx, nidx):
    mesh = plsc.VectorSubcoreMesh(core_axis_name="c", subcore_axis_name="s")

    @functools.partial(
        pl.kernel,
        mesh=mesh,
        out_type=[
            jax.ShapeDtypeStruct((M, D), jnp.float32),
            jax.ShapeDtypeStruct((M, D), jnp.float32),
            jax.ShapeDtypeStruct((M,), jnp.float32),
        ],
        scratch_types=[
            pltpu.VMEM((GCH,), jnp.int32),
            pltpu.VMEM((GCH,), jnp.int32),
            pltpu.VMEM((GCH, D), jnp.float32),
            pltpu.VMEM((GCH, D), jnp.float32),
            pltpu.VMEM((GCH,), jnp.float32),
            pltpu.SemaphoreType.DMA,
            pltpu.SemaphoreType.DMA,
            pltpu.SemaphoreType.DMA,
        ],
    )
    def k(x_hbm, w_hbm, si_hbm, ni_hbm, sout, nout, wout,
          si_v, ni_v, srow, nrow, wrow, sem0, sem1, sem2):
        wid = lax.axis_index("c") * NS + lax.axis_index("s")

        def body(j, carry):
            base = wid * EPW + j * GCH
            pltpu.sync_copy(si_hbm.at[pl.ds(base, GCH)], si_v)
            pltpu.sync_copy(ni_hbm.at[pl.ds(base, GCH)], ni_v)
            c0 = pltpu.async_copy(x_hbm.at[si_v], srow, sem0)
            c1 = pltpu.async_copy(x_hbm.at[ni_v], nrow, sem1)
            c2 = pltpu.async_copy(w_hbm.at[ni_v], wrow, sem2)
            c0.wait()
            c1.wait()
            c2.wait()
            pltpu.sync_copy(srow, sout.at[pl.ds(base, GCH)])
            pltpu.sync_copy(nrow, nout.at[pl.ds(base, GCH)])
            pltpu.sync_copy(wrow, wout.at[pl.ds(base, GCH)])
            return carry

        lax.fori_loop(0, EPW // GCH, body, 0)

    return k(x, w, sidx, nidx)


def _dense_tc(sfea, nfea, w3, si3, w0s, w0n, b0r, w1g, mw1s, mb1p, misc):
    nblk = M // EB

    def body(sref, nref, wref, siref, w0s_r, w0n_r, b0_r, w1g_r, m1_r, mb1_r,
             mi_r, cout, vout):
        s = sref[...]
        n = nref[...]
        h = lax.dot_general(s, w0s_r[...], (((1,), (1,)), ((), ())),
                            preferred_element_type=jnp.float32)
        h = h + lax.dot_general(n, w0n_r[...], (((1,), (1,)), ((), ())),
                                preferred_element_type=jnp.float32)
        h = h + b0_r[...][0][None, :]
        h = jnp.where(h >= 0, h, 0.01 * h)
        g8 = lax.dot_general(h[:, :H * HID], w1g_r[...],
                             (((1,), (0,)), ((), ())),
                             preferred_element_type=jnp.float32)
        g8 = g8 + mi_r[...][1][None, :]
        logw = jnp.log(wref[...][0, 0, :])
        c8 = jnp.exp(g8 + logw[:, None] * mi_r[...][0][None, :])
        outs = []
        for hh in range(H):
            hm = h[:, H * HID + hh * HID:H * HID + (hh + 1) * HID]
            mh = lax.dot_general(hm, m1_r[...][hh * HID:(hh + 1) * HID, :],
                                 (((1,), (0,)), ((), ())),
                                 preferred_element_type=jnp.float32)
            mh = mh + mb1_r[...][hh][None, :]
            outs.append(c8[:, hh:hh + 1] * mh)
        cout[...] = jnp.concatenate(outs, axis=1)
        si = siref[...][0, 0, :]
        base = (lax.rem(si, jnp.full((EB,), 8, jnp.int32)) * 16)[:, None]
        colidx = lax.broadcasted_iota(jnp.int32, (EB, 128), 1)
        cpos = jnp.zeros((EB, 128), jnp.float32)
        for hh in range(H):
            cpos = cpos + jnp.where(colidx == base + hh,
                                    c8[:, hh:hh + 1], 0.0)
        vout[...] = cpos

    full = lambda shape: pl.BlockSpec(shape, lambda i: tuple(0 for _ in shape))
    return pl.pallas_call(
        body,
        grid=(nblk,),
        in_specs=[
            pl.BlockSpec((EB, D), lambda i: (i, 0)),
            pl.BlockSpec((EB, D), lambda i: (i, 0)),
            pl.BlockSpec((1, 1, EB), lambda i: (i, 0, 0)),
            pl.BlockSpec((1, 1, EB), lambda i: (i, 0, 0)),
            full((F1, D)),
            full((F1, D)),
            full((8, F1)),
            full((H * HID, 8)),
            full((H * HID, D)),
            full((8, D)),
            full((8, 8)),
        ],
        out_specs=[
            pl.BlockSpec((EB, CW), lambda i: (i, 0)),
            pl.BlockSpec((EB, 128), lambda i: (i, 0)),
        ],
        out_shape=[
            jax.ShapeDtypeStruct((M, CW), jnp.float32),
            jax.ShapeDtypeStruct((M, 128), jnp.float32),
        ],
    )(sfea, nfea, w3, si3, w0s, w0n, b0r, w1g, mw1s, mb1p, misc)


def _scatter_sc(contrib, cvals, sidx, zacc, zden):
    mesh = plsc.VectorSubcoreMesh(core_axis_name="c", subcore_axis_name="s")

    @functools.partial(
        pl.kernel,
        mesh=mesh,
        out_type=[
            jax.ShapeDtypeStruct((NC, H, NPAD, D), jnp.float32),
            jax.ShapeDtypeStruct((NC, DNR, 128), jnp.float32),
        ],
        scratch_types=[
            pltpu.VMEM_SHARED((NPAD, D), jnp.float32),
            pltpu.VMEM_SHARED((DNR, 128), jnp.float32),
            pltpu.VMEM((SCH,), jnp.int32),
            pltpu.VMEM((SCH,), jnp.int32),
            pltpu.VMEM((SCH, D), jnp.float32),
            pltpu.VMEM((SCH, 128), jnp.float32),
        ],
    )
    def k(c_hbm, v_hbm, si_hbm, za_hbm, zd_hbm, aout, dout,
          acc_sh, den_sh, si_v, l8_v, cbuf, vbuf):
        c = lax.axis_index("c")
        s = lax.axis_index("s")
        wid = c * NS + s
        a0 = s * APT
        d0 = s * DPT
        pltpu.sync_copy(zd_hbm.at[pl.ds(d0, DPT)], den_sh.at[pl.ds(d0, DPT)])
        eight = jnp.full((16,), 8, jnp.int32)

        for hp in range(H):
            pltpu.sync_copy(za_hbm.at[pl.ds(a0, APT)],
                            acc_sh.at[pl.ds(a0, APT)])
            plsc.subcore_barrier()

            def body(j, carry, hp=hp):
                e0 = wid * EPW + j * SCH
                pltpu.sync_copy(si_hbm.at[pl.ds(e0, SCH)], si_v)
                pltpu.sync_copy(c_hbm.at[pl.ds(e0, SCH), pl.ds(hp * D, D)],
                                cbuf)
                if hp == 0:
                    pltpu.sync_copy(v_hbm.at[pl.ds(e0, SCH)], vbuf)
                    for kk in range(SCH // 16):
                        v = si_v[pl.ds(kk * 16, 16)]
                        l8_v[pl.ds(kk * 16, 16)] = lax.div(v, eight)
                pltpu.sync_copy(cbuf, acc_sh.at[si_v], add=True)
                if hp == 0:
                    pltpu.sync_copy(vbuf, den_sh.at[l8_v], add=True)
                return carry

            lax.fori_loop(0, EPW // SCH, body, 0)
            plsc.subcore_barrier()
            pltpu.sync_copy(acc_sh.at[pl.ds(a0, APT)],
                            aout.at[c, hp, pl.ds(a0, APT)])
        pltpu.sync_copy(den_sh.at[pl.ds(d0, DPT)], dout.at[c, pl.ds(d0, DPT)])

    return k(contrib, cvals, sidx, zacc, zden)


def _combine_tc(acc, den, x):
    NB = 1000

    def body(aref, dref, xref, oref):
        a = aref[...]
        d = dref[...][0] + dref[...][1]
        tot = xref[...]
        for hh in range(H):
            num = a[0, hh] + a[1, hh]
            tot = tot + (num / (d[:, hh:hh + 1] + 1e-10)) * (1.0 / H)
        oref[...] = tot

    return pl.pallas_call(
        body,
        grid=(N // NB,),
        in_specs=[
            pl.BlockSpec((NC, H, NB, D), lambda i: (0, 0, i, 0)),
            pl.BlockSpec((NC, NB, VW), lambda i: (0, i, 0)),
            pl.BlockSpec((NB, D), lambda i: (i, 0)),
        ],
        out_specs=pl.BlockSpec((NB, D), lambda i: (i, 0)),
        out_shape=jax.ShapeDtypeStruct((N, D), jnp.float32),
    )(acc, den, x)


def kernel(elem_weights, elem_in_fea, self_fea_idx, nbr_fea_idx,
           gate_w0, gate_b0, gate_w1, gate_b1,
           msg_w0, msg_b0, msg_w1, msg_b1, pow_p):
    w1d = elem_weights.reshape(N)
    sfea, nfea, wg = _gather_sc(elem_in_fea, w1d, self_fea_idx, nbr_fea_idx)

    w0s = jnp.concatenate([gate_w0[:, :, :D].reshape(H * HID, D),
                           msg_w0[:, :, :D].reshape(H * HID, D)], axis=0)
    w0n = jnp.concatenate([gate_w0[:, :, D:].reshape(H * HID, D),
                           msg_w0[:, :, D:].reshape(H * HID, D)], axis=0)
    b0 = jnp.concatenate([gate_b0.reshape(-1), msg_b0.reshape(-1)])
    b0r = jnp.broadcast_to(b0[None, :], (8, F1))
    w1g = jnp.zeros((H * HID, 8), jnp.float32)
    for hh in range(H):
        w1g = w1g.at[hh * HID:(hh + 1) * HID, hh].set(gate_w1[hh, 0])
    mw1s = jnp.transpose(msg_w1, (0, 2, 1)).reshape(H * HID, D)
    mb1p = jnp.concatenate([msg_b1, jnp.zeros((8 - H, D), jnp.float32)], axis=0)
    pow8 = jnp.concatenate([pow_p, jnp.zeros((8 - H,), jnp.float32)])
    b1g8 = jnp.concatenate([gate_b1[:, 0], jnp.zeros((8 - H,), jnp.float32)])
    misc = jnp.stack([pow8, b1g8] + [jnp.zeros((8,), jnp.float32)] * 6)

    w3 = wg.reshape(M // EB, 1, EB)
    si3 = self_fea_idx.reshape(M // EB, 1, EB)
    contrib, cvals = _dense_tc(sfea, nfea, w3, si3, w0s, w0n, b0r, w1g, mw1s,
                               mb1p, misc)

    zacc = jnp.zeros((NPAD, D), jnp.float32)
    zden = jnp.zeros((DNR, 128), jnp.float32)
    acc, denp = _scatter_sc(contrib, cvals, self_fea_idx, zacc, zden)
    den = denp.reshape(NC, NPAD, VW)

    return _combine_tc(acc, den, elem_in_fea)


# speedup vs baseline: 8.0176x; 1.0187x over previous
import functools

import jax
import jax.numpy as jnp
from jax import lax
from jax.experimental import pallas as pl
from jax.experimental.pallas import tpu as pltpu
from jax.experimental.pallas import tpu_sc as plsc

N = 10000
M = 320000
D = 128
H = 3
HID = 256

NC = 2
NS = 16
NW = NC * NS

EPW = M // NW
GCH = 80
SCH = 80
NPAD = 10240
APT = NPAD // NS
DNR = NPAD // 8
DPT = DNR // NS
CW = H * D
VW = 16
F1 = 2 * H * HID
EB = 512


def _gather_sc(x, w, sidx, nidx):
    mesh = plsc.VectorSubcoreMesh(core_axis_name="c", subcore_axis_name="s")

    @functools.partial(
        pl.kernel,
        mesh=mesh,
        out_type=[
            jax.ShapeDtypeStruct((M, D), jnp.float32),
            jax.ShapeDtypeStruct((M, D), jnp.float32),
            jax.ShapeDtypeStruct((M,), jnp.float32),
        ],
        scratch_types=[
            pltpu.VMEM((GCH,), jnp.int32),
            pltpu.VMEM((GCH,), jnp.int32),
            pltpu.VMEM((GCH, D), jnp.float32),
            pltpu.VMEM((GCH, D), jnp.float32),
            pltpu.VMEM((GCH,), jnp.float32),
            pltpu.SemaphoreType.DMA,
            pltpu.SemaphoreType.DMA,
            pltpu.SemaphoreType.DMA,
        ],
    )
    def k(x_hbm, w_hbm, si_hbm, ni_hbm, sout, nout, wout,
          si_v, ni_v, srow, nrow, wrow, sem0, sem1, sem2):
        wid = lax.axis_index("c") * NS + lax.axis_index("s")

        def body(j, carry):
            base = wid * EPW + j * GCH
            pltpu.sync_copy(si_hbm.at[pl.ds(base, GCH)], si_v)
            pltpu.sync_copy(ni_hbm.at[pl.ds(base, GCH)], ni_v)
            c0 = pltpu.async_copy(x_hbm.at[si_v], srow, sem0)
            c1 = pltpu.async_copy(x_hbm.at[ni_v], nrow, sem1)
            c2 = pltpu.async_copy(w_hbm.at[ni_v], wrow, sem2)
            c0.wait()
            c1.wait()
            c2.wait()
            pltpu.sync_copy(srow, sout.at[pl.ds(base, GCH)])
            pltpu.sync_copy(nrow, nout.at[pl.ds(base, GCH)])
            pltpu.sync_copy(wrow, wout.at[pl.ds(base, GCH)])
            return carry

        lax.fori_loop(0, EPW // GCH, body, 0)

    return k(x, w, sidx, nidx)


def _dense_tc(sfea, nfea, w3, si3, w0s, w0n, b0r, w1g, mw1s, mb1p, misc):
    nblk = M // EB

    def body(sref, nref, wref, siref, w0s_r, w0n_r, b0_r, w1g_r, m1_r, mb1_r,
             mi_r, cout, vout):
        s = sref[...].astype(jnp.bfloat16)
        n = nref[...].astype(jnp.bfloat16)
        h = lax.dot_general(s, w0s_r[...], (((1,), (1,)), ((), ())),
                            preferred_element_type=jnp.float32)
        h = h + lax.dot_general(n, w0n_r[...], (((1,), (1,)), ((), ())),
                                preferred_element_type=jnp.float32)
        h = h + b0_r[...][0][None, :]
        h = jnp.where(h >= 0, h, 0.01 * h)
        hb = h.astype(jnp.bfloat16)
        g8 = lax.dot_general(hb[:, :H * HID], w1g_r[...],
                             (((1,), (0,)), ((), ())),
                             preferred_element_type=jnp.float32)
        g8 = g8 + mi_r[...][1][None, :]
        logw = jnp.log(wref[...][0, 0, :])
        c8 = jnp.exp(g8 + logw[:, None] * mi_r[...][0][None, :])
        outs = []
        for hh in range(H):
            hm = hb[:, H * HID + hh * HID:H * HID + (hh + 1) * HID]
            mh = lax.dot_general(hm, m1_r[...][hh * HID:(hh + 1) * HID, :],
                                 (((1,), (0,)), ((), ())),
                                 preferred_element_type=jnp.float32)
            mh = mh + mb1_r[...][hh][None, :]
            outs.append(c8[:, hh:hh + 1] * mh)
        cout[...] = jnp.concatenate(outs, axis=1)
        si = siref[...][0, 0, :]
        base = (lax.rem(si, jnp.full((EB,), 8, jnp.int32)) * 16)[:, None]
        colidx = lax.broadcasted_iota(jnp.int32, (EB, 128), 1)
        cpos = jnp.zeros((EB, 128), jnp.float32)
        for hh in range(H):
            cpos = cpos + jnp.where(colidx == base + hh,
                                    c8[:, hh:hh + 1], 0.0)
        vout[...] = cpos

    full = lambda shape: pl.BlockSpec(shape, lambda i: tuple(0 for _ in shape))
    return pl.pallas_call(
        body,
        grid=(nblk,),
        in_specs=[
            pl.BlockSpec((EB, D), lambda i: (i, 0)),
            pl.BlockSpec((EB, D), lambda i: (i, 0)),
            pl.BlockSpec((1, 1, EB), lambda i: (i, 0, 0)),
            pl.BlockSpec((1, 1, EB), lambda i: (i, 0, 0)),
            full((F1, D)),
            full((F1, D)),
            full((8, F1)),
            full((H * HID, 8)),
            full((H * HID, D)),
            full((8, D)),
            full((8, 8)),
        ],
        out_specs=[
            pl.BlockSpec((EB, CW), lambda i: (i, 0)),
            pl.BlockSpec((EB, 128), lambda i: (i, 0)),
        ],
        out_shape=[
            jax.ShapeDtypeStruct((M, CW), jnp.float32),
            jax.ShapeDtypeStruct((M, 128), jnp.float32),
        ],
    )(sfea, nfea, w3, si3, w0s, w0n, b0r, w1g, mw1s, mb1p, misc)


def _scatter_sc(contrib, cvals, sidx, zacc, zden):
    mesh = plsc.VectorSubcoreMesh(core_axis_name="c", subcore_axis_name="s")

    @functools.partial(
        pl.kernel,
        mesh=mesh,
        out_type=[
            jax.ShapeDtypeStruct((NC, H, NPAD, D), jnp.float32),
            jax.ShapeDtypeStruct((NC, DNR, 128), jnp.float32),
        ],
        scratch_types=[
            pltpu.VMEM_SHARED((NPAD, D), jnp.float32),
            pltpu.VMEM_SHARED((DNR, 128), jnp.float32),
            pltpu.VMEM((SCH,), jnp.int32),
            pltpu.VMEM((SCH,), jnp.int32),
            pltpu.VMEM((SCH, D), jnp.float32),
            pltpu.VMEM((SCH, 128), jnp.float32),
        ],
    )
    def k(c_hbm, v_hbm, si_hbm, za_hbm, zd_hbm, aout, dout,
          acc_sh, den_sh, si_v, l8_v, cbuf, vbuf):
        c = lax.axis_index("c")
        s = lax.axis_index("s")
        wid = c * NS + s
        a0 = s * APT
        d0 = s * DPT
        pltpu.sync_copy(zd_hbm.at[pl.ds(d0, DPT)], den_sh.at[pl.ds(d0, DPT)])
        eight = jnp.full((16,), 8, jnp.int32)

        for hp in range(H):
            pltpu.sync_copy(za_hbm.at[pl.ds(a0, APT)],
                            acc_sh.at[pl.ds(a0, APT)])
            plsc.subcore_barrier()

            def body(j, carry, hp=hp):
                e0 = wid * EPW + j * SCH
                pltpu.sync_copy(si_hbm.at[pl.ds(e0, SCH)], si_v)
                pltpu.sync_copy(c_hbm.at[pl.ds(e0, SCH), pl.ds(hp * D, D)],
                                cbuf)
                if hp == 0:
                    pltpu.sync_copy(v_hbm.at[pl.ds(e0, SCH)], vbuf)
                    for kk in range(SCH // 16):
                        v = si_v[pl.ds(kk * 16, 16)]
                        l8_v[pl.ds(kk * 16, 16)] = lax.div(v, eight)
                pltpu.sync_copy(cbuf, acc_sh.at[si_v], add=True)
                if hp == 0:
                    pltpu.sync_copy(vbuf, den_sh.at[l8_v], add=True)
                return carry

            lax.fori_loop(0, EPW // SCH, body, 0)
            plsc.subcore_barrier()
            pltpu.sync_copy(acc_sh.at[pl.ds(a0, APT)],
                            aout.at[c, hp, pl.ds(a0, APT)])
        pltpu.sync_copy(den_sh.at[pl.ds(d0, DPT)], dout.at[c, pl.ds(d0, DPT)])

    return k(contrib, cvals, sidx, zacc, zden)


def _combine_tc(acc, den, x):
    NB = 1000

    def body(aref, dref, xref, oref):
        a = aref[...]
        d = dref[...][0] + dref[...][1]
        tot = xref[...]
        for hh in range(H):
            num = a[0, hh] + a[1, hh]
            tot = tot + (num / (d[:, hh:hh + 1] + 1e-10)) * (1.0 / H)
        oref[...] = tot

    return pl.pallas_call(
        body,
        grid=(N // NB,),
        in_specs=[
            pl.BlockSpec((NC, H, NB, D), lambda i: (0, 0, i, 0)),
            pl.BlockSpec((NC, NB, VW), lambda i: (0, i, 0)),
            pl.BlockSpec((NB, D), lambda i: (i, 0)),
        ],
        out_specs=pl.BlockSpec((NB, D), lambda i: (i, 0)),
        out_shape=jax.ShapeDtypeStruct((N, D), jnp.float32),
    )(acc, den, x)


def kernel(elem_weights, elem_in_fea, self_fea_idx, nbr_fea_idx,
           gate_w0, gate_b0, gate_w1, gate_b1,
           msg_w0, msg_b0, msg_w1, msg_b1, pow_p):
    w1d = elem_weights.reshape(N)
    sfea, nfea, wg = _gather_sc(elem_in_fea, w1d, self_fea_idx, nbr_fea_idx)

    w0s = jnp.concatenate([gate_w0[:, :, :D].reshape(H * HID, D),
                           msg_w0[:, :, :D].reshape(H * HID, D)],
                          axis=0).astype(jnp.bfloat16)
    w0n = jnp.concatenate([gate_w0[:, :, D:].reshape(H * HID, D),
                           msg_w0[:, :, D:].reshape(H * HID, D)],
                          axis=0).astype(jnp.bfloat16)
    b0 = jnp.concatenate([gate_b0.reshape(-1), msg_b0.reshape(-1)])
    b0r = jnp.broadcast_to(b0[None, :], (8, F1))
    w1g = jnp.zeros((H * HID, 8), jnp.float32)
    for hh in range(H):
        w1g = w1g.at[hh * HID:(hh + 1) * HID, hh].set(gate_w1[hh, 0])
    w1g = w1g.astype(jnp.bfloat16)
    mw1s = jnp.transpose(msg_w1, (0, 2, 1)).reshape(H * HID, D)
    mw1s = mw1s.astype(jnp.bfloat16)
    mb1p = jnp.concatenate([msg_b1, jnp.zeros((8 - H, D), jnp.float32)], axis=0)
    pow8 = jnp.concatenate([pow_p, jnp.zeros((8 - H,), jnp.float32)])
    b1g8 = jnp.concatenate([gate_b1[:, 0], jnp.zeros((8 - H,), jnp.float32)])
    misc = jnp.stack([pow8, b1g8] + [jnp.zeros((8,), jnp.float32)] * 6)

    w3 = wg.reshape(M // EB, 1, EB)
    si3 = self_fea_idx.reshape(M // EB, 1, EB)
    contrib, cvals = _dense_tc(sfea, nfea, w3, si3, w0s, w0n, b0r, w1g, mw1s,
                               mb1p, misc)

    zacc = jnp.zeros((NPAD, D), jnp.float32)
    zden = jnp.zeros((DNR, 128), jnp.float32)
    acc, denp = _scatter_sc(contrib, cvals, self_fea_idx, zacc, zden)
    den = denp.reshape(NC, NPAD, VW)

    return _combine_tc(acc, den, elem_in_fea)


# speedup vs baseline: 9.6186x; 1.1997x over previous
import functools

import jax
import jax.numpy as jnp
from jax import lax
from jax.experimental import pallas as pl
from jax.experimental.pallas import tpu as pltpu
from jax.experimental.pallas import tpu_sc as plsc

N = 10000
M = 320000
D = 128
H = 3
HID = 256

NC = 2
NS = 16
NW = NC * NS

EPW = M // NW
GCH = 80
SCH = 80
NPAD = 10240
APT = NPAD // NS
DNR = NPAD // 8
DPT = DNR // NS
CW = H * D
VW = 16
F1 = 2 * H * HID
EB = 512


def _gather_sc(x, w, sidx, nidx):
    mesh = plsc.VectorSubcoreMesh(core_axis_name="c", subcore_axis_name="s")

    @functools.partial(
        pl.kernel,
        mesh=mesh,
        out_type=[
            jax.ShapeDtypeStruct((M, D), jnp.float32),
            jax.ShapeDtypeStruct((M, D), jnp.float32),
            jax.ShapeDtypeStruct((M,), jnp.float32),
        ],
        scratch_types=[
            pltpu.VMEM((GCH,), jnp.int32),
            pltpu.VMEM((GCH,), jnp.int32),
            pltpu.VMEM((GCH, D), jnp.float32),
            pltpu.VMEM((GCH, D), jnp.float32),
            pltpu.VMEM((GCH,), jnp.float32),
            pltpu.SemaphoreType.DMA,
            pltpu.SemaphoreType.DMA,
            pltpu.SemaphoreType.DMA,
        ],
    )
    def k(x_hbm, w_hbm, si_hbm, ni_hbm, sout, nout, wout,
          si_v, ni_v, srow, nrow, wrow, sem0, sem1, sem2):
        wid = lax.axis_index("c") * NS + lax.axis_index("s")

        def body(j, carry):
            base = wid * EPW + j * GCH
            pltpu.sync_copy(si_hbm.at[pl.ds(base, GCH)], si_v)
            pltpu.sync_copy(ni_hbm.at[pl.ds(base, GCH)], ni_v)
            c0 = pltpu.async_copy(x_hbm.at[si_v], srow, sem0)
            c1 = pltpu.async_copy(x_hbm.at[ni_v], nrow, sem1)
            c2 = pltpu.async_copy(w_hbm.at[ni_v], wrow, sem2)
            c0.wait()
            c1.wait()
            c2.wait()
            pltpu.sync_copy(srow, sout.at[pl.ds(base, GCH)])
            pltpu.sync_copy(nrow, nout.at[pl.ds(base, GCH)])
            pltpu.sync_copy(wrow, wout.at[pl.ds(base, GCH)])
            return carry

        lax.fori_loop(0, EPW // GCH, body, 0)

    return k(x, w, sidx, nidx)


def _dense_tc(sfea, nfea, w3, si3, w0s, w0n, b0r, w1g, mw1s, mb1p, misc):
    nblk = M // EB

    def body(sref, nref, wref, siref, w0s_r, w0n_r, b0_r, w1g_r, m1_r, mb1_r,
             mi_r, cout, vout):
        s = sref[...].astype(jnp.bfloat16)
        n = nref[...].astype(jnp.bfloat16)
        h = lax.dot_general(s, w0s_r[...], (((1,), (1,)), ((), ())),
                            preferred_element_type=jnp.float32)
        h = h + lax.dot_general(n, w0n_r[...], (((1,), (1,)), ((), ())),
                                preferred_element_type=jnp.float32)
        h = h + b0_r[...][0][None, :]
        h = jnp.where(h >= 0, h, 0.01 * h)
        hb = h.astype(jnp.bfloat16)
        g8 = lax.dot_general(hb[:, :H * HID], w1g_r[...],
                             (((1,), (0,)), ((), ())),
                             preferred_element_type=jnp.float32)
        g8 = g8 + mi_r[...][1][None, :]
        logw = jnp.log(wref[...][0, 0, :])
        c8 = jnp.exp(g8 + logw[:, None] * mi_r[...][0][None, :])
        outs = []
        for hh in range(H):
            hm = hb[:, H * HID + hh * HID:H * HID + (hh + 1) * HID]
            mh = lax.dot_general(hm, m1_r[...][hh * HID:(hh + 1) * HID, :],
                                 (((1,), (0,)), ((), ())),
                                 preferred_element_type=jnp.float32)
            mh = mh + mb1_r[...][hh][None, :]
            outs.append(c8[:, hh:hh + 1] * mh)
        cout[...] = jnp.concatenate(outs, axis=1)
        si = siref[...][0, 0, :]
        base = (lax.rem(si, jnp.full((EB,), 8, jnp.int32)) * 16)[:, None]
        colidx = lax.broadcasted_iota(jnp.int32, (EB, 128), 1)
        cpos = jnp.zeros((EB, 128), jnp.float32)
        for hh in range(H):
            cpos = cpos + jnp.where(colidx == base + hh,
                                    c8[:, hh:hh + 1], 0.0)
        vout[...] = cpos

    full = lambda shape: pl.BlockSpec(shape, lambda i: tuple(0 for _ in shape))
    return pl.pallas_call(
        body,
        grid=(nblk,),
        in_specs=[
            pl.BlockSpec((EB, D), lambda i: (i, 0)),
            pl.BlockSpec((EB, D), lambda i: (i, 0)),
            pl.BlockSpec((1, 1, EB), lambda i: (i, 0, 0)),
            pl.BlockSpec((1, 1, EB), lambda i: (i, 0, 0)),
            full((F1, D)),
            full((F1, D)),
            full((8, F1)),
            full((H * HID, 8)),
            full((H * HID, D)),
            full((8, D)),
            full((8, 8)),
        ],
        out_specs=[
            pl.BlockSpec((EB, CW), lambda i: (i, 0)),
            pl.BlockSpec((EB, 128), lambda i: (i, 0)),
        ],
        out_shape=[
            jax.ShapeDtypeStruct((M, CW), jnp.float32),
            jax.ShapeDtypeStruct((M, 128), jnp.float32),
        ],
    )(sfea, nfea, w3, si3, w0s, w0n, b0r, w1g, mw1s, mb1p, misc)


def _scatter_sc(contrib, cvals, sidx, zacc, zden):
    mesh = plsc.VectorSubcoreMesh(core_axis_name="c", subcore_axis_name="s")

    @functools.partial(
        pl.kernel,
        mesh=mesh,
        out_type=[
            jax.ShapeDtypeStruct((NC, H, NPAD, D), jnp.float32),
            jax.ShapeDtypeStruct((NC, DNR, 128), jnp.float32),
        ],
        scratch_types=[
            pltpu.VMEM_SHARED((NPAD, D), jnp.float32),
            pltpu.VMEM_SHARED((DNR, 128), jnp.float32),
            pltpu.VMEM((SCH,), jnp.int32),
            pltpu.VMEM((SCH,), jnp.int32),
            pltpu.VMEM((SCH,), jnp.int32),
            pltpu.VMEM((SCH, D), jnp.float32),
            pltpu.VMEM((SCH, D), jnp.float32),
            pltpu.VMEM((SCH, 128), jnp.float32),
            pltpu.SemaphoreType.DMA,
            pltpu.SemaphoreType.DMA,
            pltpu.SemaphoreType.DMA,
            pltpu.SemaphoreType.DMA,
        ],
    )
    def k(c_hbm, v_hbm, si_hbm, za_hbm, zd_hbm, aout, dout,
          acc_sh, den_sh, si_a, si_b, l8_v, cb_a, cb_b, vbuf,
          sem_sa, sem_sb, sem_ca, sem_cb):
        c = lax.axis_index("c")
        s = lax.axis_index("s")
        wid = c * NS + s
        a0 = s * APT
        d0 = s * DPT
        pltpu.sync_copy(zd_hbm.at[pl.ds(d0, DPT)], den_sh.at[pl.ds(d0, DPT)])
        eight = jnp.full((16,), 8, jnp.int32)
        nch = EPW // SCH

        for hp in range(H):
            col = pl.ds(hp * D, D)

            def start(j, sib, cb, sem_s, sem_c):
                e0 = wid * EPW + j * SCH
                pltpu.async_copy(si_hbm.at[pl.ds(e0, SCH)], sib, sem_s)
                pltpu.async_copy(c_hbm.at[pl.ds(e0, SCH), col], cb, sem_c)

            def wait(j, sib, cb, sem_s, sem_c):
                e0 = wid * EPW + j * SCH
                pltpu.make_async_copy(si_hbm.at[pl.ds(e0, SCH)], sib,
                                      sem_s).wait()
                pltpu.make_async_copy(c_hbm.at[pl.ds(e0, SCH), col], cb,
                                      sem_c).wait()

            def process(j, sib, cb, hp=hp):
                if hp == 0:
                    e0 = wid * EPW + j * SCH
                    pltpu.sync_copy(v_hbm.at[pl.ds(e0, SCH)], vbuf)
                    for kk in range(SCH // 16):
                        v = sib[pl.ds(kk * 16, 16)]
                        l8_v[pl.ds(kk * 16, 16)] = lax.div(v, eight)
                pltpu.sync_copy(cb, acc_sh.at[sib], add=True)
                if hp == 0:
                    pltpu.sync_copy(vbuf, den_sh.at[l8_v], add=True)

            pltpu.sync_copy(za_hbm.at[pl.ds(a0, APT)],
                            acc_sh.at[pl.ds(a0, APT)])
            plsc.subcore_barrier()
            start(0, si_a, cb_a, sem_sa, sem_ca)

            def body(g, carry):
                j0 = 2 * g
                start(j0 + 1, si_b, cb_b, sem_sb, sem_cb)
                wait(j0, si_a, cb_a, sem_sa, sem_ca)
                process(j0, si_a, cb_a)
                start(j0 + 2, si_a, cb_a, sem_sa, sem_ca)
                wait(j0 + 1, si_b, cb_b, sem_sb, sem_cb)
                process(j0 + 1, si_b, cb_b)
                return carry

            lax.fori_loop(0, (nch - 1) // 2, body, 0)
            wait(nch - 1, si_a, cb_a, sem_sa, sem_ca)
            process(nch - 1, si_a, cb_a)
            plsc.subcore_barrier()
            pltpu.sync_copy(acc_sh.at[pl.ds(a0, APT)],
                            aout.at[c, hp, pl.ds(a0, APT)])
        pltpu.sync_copy(den_sh.at[pl.ds(d0, DPT)], dout.at[c, pl.ds(d0, DPT)])

    return k(contrib, cvals, sidx, zacc, zden)


def _combine_tc(acc, den, x):
    NB = 1000

    def body(aref, dref, xref, oref):
        a = aref[...]
        d = dref[...][0] + dref[...][1]
        tot = xref[...]
        for hh in range(H):
            num = a[0, hh] + a[1, hh]
            tot = tot + (num / (d[:, hh:hh + 1] + 1e-10)) * (1.0 / H)
        oref[...] = tot

    return pl.pallas_call(
        body,
        grid=(N // NB,),
        in_specs=[
            pl.BlockSpec((NC, H, NB, D), lambda i: (0, 0, i, 0)),
            pl.BlockSpec((NC, NB, VW), lambda i: (0, i, 0)),
            pl.BlockSpec((NB, D), lambda i: (i, 0)),
        ],
        out_specs=pl.BlockSpec((NB, D), lambda i: (i, 0)),
        out_shape=jax.ShapeDtypeStruct((N, D), jnp.float32),
    )(acc, den, x)


def kernel(elem_weights, elem_in_fea, self_fea_idx, nbr_fea_idx,
           gate_w0, gate_b0, gate_w1, gate_b1,
           msg_w0, msg_b0, msg_w1, msg_b1, pow_p):
    w1d = elem_weights.reshape(N)
    sfea, nfea, wg = _gather_sc(elem_in_fea, w1d, self_fea_idx, nbr_fea_idx)

    w0s = jnp.concatenate([gate_w0[:, :, :D].reshape(H * HID, D),
                           msg_w0[:, :, :D].reshape(H * HID, D)],
                          axis=0).astype(jnp.bfloat16)
    w0n = jnp.concatenate([gate_w0[:, :, D:].reshape(H * HID, D),
                           msg_w0[:, :, D:].reshape(H * HID, D)],
                          axis=0).astype(jnp.bfloat16)
    b0 = jnp.concatenate([gate_b0.reshape(-1), msg_b0.reshape(-1)])
    b0r = jnp.broadcast_to(b0[None, :], (8, F1))
    w1g = jnp.zeros((H * HID, 8), jnp.float32)
    for hh in range(H):
        w1g = w1g.at[hh * HID:(hh + 1) * HID, hh].set(gate_w1[hh, 0])
    w1g = w1g.astype(jnp.bfloat16)
    mw1s = jnp.transpose(msg_w1, (0, 2, 1)).reshape(H * HID, D)
    mw1s = mw1s.astype(jnp.bfloat16)
    mb1p = jnp.concatenate([msg_b1, jnp.zeros((8 - H, D), jnp.float32)], axis=0)
    pow8 = jnp.concatenate([pow_p, jnp.zeros((8 - H,), jnp.float32)])
    b1g8 = jnp.concatenate([gate_b1[:, 0], jnp.zeros((8 - H,), jnp.float32)])
    misc = jnp.stack([pow8, b1g8] + [jnp.zeros((8,), jnp.float32)] * 6)

    w3 = wg.reshape(M // EB, 1, EB)
    si3 = self_fea_idx.reshape(M // EB, 1, EB)
    contrib, cvals = _dense_tc(sfea, nfea, w3, si3, w0s, w0n, b0r, w1g, mw1s,
                               mb1p, misc)

    zacc = jnp.zeros((NPAD, D), jnp.float32)
    zden = jnp.zeros((DNR, 128), jnp.float32)
    acc, denp = _scatter_sc(contrib, cvals, self_fea_idx, zacc, zden)
    den = denp.reshape(NC, NPAD, VW)

    return _combine_tc(acc, den, elem_in_fea)


# speedup vs baseline: 10.3326x; 1.0742x over previous
import functools

import jax
import jax.numpy as jnp
from jax import lax
from jax.experimental import pallas as pl
from jax.experimental.pallas import tpu as pltpu
from jax.experimental.pallas import tpu_sc as plsc

N = 10000
M = 320000
D = 128
H = 3
HID = 256

NC = 2
NS = 16
NW = NC * NS

EPW = M // NW
GCH = 80
SCH = 80
NPAD = 10240
APT = NPAD // NS
DNR = NPAD // 8
DPT = DNR // NS
CW = H * D
VW = 16
F1 = 2 * H * HID
EB = 512


def _gather_sc(x, w, sidx, nidx):
    mesh = plsc.VectorSubcoreMesh(core_axis_name="c", subcore_axis_name="s")

    @functools.partial(
        pl.kernel,
        mesh=mesh,
        out_type=[
            jax.ShapeDtypeStruct((M, D), jnp.float32),
            jax.ShapeDtypeStruct((M, D), jnp.float32),
            jax.ShapeDtypeStruct((M,), jnp.float32),
        ],
        scratch_types=[
            pltpu.VMEM((GCH,), jnp.int32),
            pltpu.VMEM((GCH,), jnp.int32),
            pltpu.VMEM((GCH, D), jnp.float32),
            pltpu.VMEM((GCH, D), jnp.float32),
            pltpu.VMEM((GCH,), jnp.float32),
            pltpu.VMEM((GCH,), jnp.int32),
            pltpu.VMEM((GCH,), jnp.int32),
            pltpu.VMEM((GCH, D), jnp.float32),
            pltpu.VMEM((GCH, D), jnp.float32),
            pltpu.VMEM((GCH,), jnp.float32),
            pltpu.SemaphoreType.DMA,
            pltpu.SemaphoreType.DMA,
            pltpu.SemaphoreType.DMA,
            pltpu.SemaphoreType.DMA,
            pltpu.SemaphoreType.DMA,
            pltpu.SemaphoreType.DMA,
        ],
    )
    def k(x_hbm, w_hbm, si_hbm, ni_hbm, sout, nout, wout,
          si_a, ni_a, srow_a, nrow_a, wrow_a,
          si_b, ni_b, srow_b, nrow_b, wrow_b,
          sem_ia, sem_ga, sem_wa, sem_ib, sem_gb, sem_wb):
        wid = lax.axis_index("c") * NS + lax.axis_index("s")
        bufa = (si_a, ni_a, srow_a, nrow_a, wrow_a, sem_ia, sem_ga, sem_wa)
        bufb = (si_b, ni_b, srow_b, nrow_b, wrow_b, sem_ib, sem_gb, sem_wb)
        nch = EPW // GCH

        def idx_start(j, b):
            base = wid * EPW + j * GCH
            pltpu.async_copy(si_hbm.at[pl.ds(base, GCH)], b[0], b[5])
            pltpu.async_copy(ni_hbm.at[pl.ds(base, GCH)], b[1], b[5])

        def gathers_start(j, b):
            base = wid * EPW + j * GCH
            pltpu.make_async_copy(si_hbm.at[pl.ds(base, GCH)], b[0],
                                  b[5]).wait()
            pltpu.make_async_copy(ni_hbm.at[pl.ds(base, GCH)], b[1],
                                  b[5]).wait()
            pltpu.async_copy(x_hbm.at[b[0]], b[2], b[6])
            pltpu.async_copy(x_hbm.at[b[1]], b[3], b[6])
            pltpu.async_copy(w_hbm.at[b[1]], b[4], b[6])

        def gathers_wait(b):
            pltpu.make_async_copy(x_hbm.at[b[0]], b[2], b[6]).wait()
            pltpu.make_async_copy(x_hbm.at[b[1]], b[3], b[6]).wait()
            pltpu.make_async_copy(w_hbm.at[b[1]], b[4], b[6]).wait()

        def writes_start(j, b):
            base = wid * EPW + j * GCH
            pltpu.async_copy(b[2], sout.at[pl.ds(base, GCH)], b[7])
            pltpu.async_copy(b[3], nout.at[pl.ds(base, GCH)], b[7])
            pltpu.async_copy(b[4], wout.at[pl.ds(base, GCH)], b[7])

        def writes_wait(j, b):
            base = wid * EPW + j * GCH
            pltpu.make_async_copy(b[2], sout.at[pl.ds(base, GCH)],
                                  b[7]).wait()
            pltpu.make_async_copy(b[3], nout.at[pl.ds(base, GCH)],
                                  b[7]).wait()
            pltpu.make_async_copy(b[4], wout.at[pl.ds(base, GCH)],
                                  b[7]).wait()

        idx_start(0, bufa)
        gathers_start(0, bufa)

        def body(g, carry):
            j0 = 2 * g
            idx_start(j0 + 1, bufb)
            gathers_wait(bufa)
            writes_start(j0, bufa)
            gathers_start(j0 + 1, bufb)
            writes_wait(j0, bufa)
            idx_start(j0 + 2, bufa)
            gathers_wait(bufb)
            writes_start(j0 + 1, bufb)
            gathers_start(j0 + 2, bufa)
            writes_wait(j0 + 1, bufb)
            return carry

        lax.fori_loop(0, (nch - 1) // 2, body, 0)
        gathers_wait(bufa)
        base = wid * EPW + (nch - 1) * GCH
        pltpu.sync_copy(srow_a, sout.at[pl.ds(base, GCH)])
        pltpu.sync_copy(nrow_a, nout.at[pl.ds(base, GCH)])
        pltpu.sync_copy(wrow_a, wout.at[pl.ds(base, GCH)])

    return k(x, w, sidx, nidx)


def _dense_tc(sfea, nfea, w3, si3, w0s, w0n, b0r, w1g, mw1s, mb1p, misc):
    nblk = M // EB

    def body(sref, nref, wref, siref, w0s_r, w0n_r, b0_r, w1g_r, m1_r, mb1_r,
             mi_r, cout, vout):
        s = sref[...].astype(jnp.bfloat16)
        n = nref[...].astype(jnp.bfloat16)
        h = lax.dot_general(s, w0s_r[...], (((1,), (1,)), ((), ())),
                            preferred_element_type=jnp.float32)
        h = h + lax.dot_general(n, w0n_r[...], (((1,), (1,)), ((), ())),
                                preferred_element_type=jnp.float32)
        h = h + b0_r[...][0][None, :]
        h = jnp.where(h >= 0, h, 0.01 * h)
        hb = h.astype(jnp.bfloat16)
        g8 = lax.dot_general(hb[:, :H * HID], w1g_r[...],
                             (((1,), (0,)), ((), ())),
                             preferred_element_type=jnp.float32)
        g8 = g8 + mi_r[...][1][None, :]
        logw = jnp.log(wref[...][0, 0, :])
        c8 = jnp.exp(g8 + logw[:, None] * mi_r[...][0][None, :])
        outs = []
        for hh in range(H):
            hm = hb[:, H * HID + hh * HID:H * HID + (hh + 1) * HID]
            mh = lax.dot_general(hm, m1_r[...][hh * HID:(hh + 1) * HID, :],
                                 (((1,), (0,)), ((), ())),
                                 preferred_element_type=jnp.float32)
            mh = mh + mb1_r[...][hh][None, :]
            outs.append(c8[:, hh:hh + 1] * mh)
        cout[...] = jnp.concatenate(outs, axis=1)
        si = siref[...][0, 0, :]
        base = (lax.rem(si, jnp.full((EB,), 8, jnp.int32)) * 16)[:, None]
        colidx = lax.broadcasted_iota(jnp.int32, (EB, 128), 1)
        cpos = jnp.zeros((EB, 128), jnp.float32)
        for hh in range(H):
            cpos = cpos + jnp.where(colidx == base + hh,
                                    c8[:, hh:hh + 1], 0.0)
        vout[...] = cpos

    full = lambda shape: pl.BlockSpec(shape, lambda i: tuple(0 for _ in shape))
    return pl.pallas_call(
        body,
        grid=(nblk,),
        in_specs=[
            pl.BlockSpec((EB, D), lambda i: (i, 0)),
            pl.BlockSpec((EB, D), lambda i: (i, 0)),
            pl.BlockSpec((1, 1, EB), lambda i: (i, 0, 0)),
            pl.BlockSpec((1, 1, EB), lambda i: (i, 0, 0)),
            full((F1, D)),
            full((F1, D)),
            full((8, F1)),
            full((H * HID, 8)),
            full((H * HID, D)),
            full((8, D)),
            full((8, 8)),
        ],
        out_specs=[
            pl.BlockSpec((EB, CW), lambda i: (i, 0)),
            pl.BlockSpec((EB, 128), lambda i: (i, 0)),
        ],
        out_shape=[
            jax.ShapeDtypeStruct((M, CW), jnp.float32),
            jax.ShapeDtypeStruct((M, 128), jnp.float32),
        ],
    )(sfea, nfea, w3, si3, w0s, w0n, b0r, w1g, mw1s, mb1p, misc)


def _scatter_sc(contrib, cvals, sidx, zacc, zden):
    mesh = plsc.VectorSubcoreMesh(core_axis_name="c", subcore_axis_name="s")

    @functools.partial(
        pl.kernel,
        mesh=mesh,
        out_type=[
            jax.ShapeDtypeStruct((NC, H, NPAD, D), jnp.float32),
            jax.ShapeDtypeStruct((NC, DNR, 128), jnp.float32),
        ],
        scratch_types=[
            pltpu.VMEM_SHARED((NPAD, D), jnp.float32),
            pltpu.VMEM_SHARED((DNR, 128), jnp.float32),
            pltpu.VMEM((SCH,), jnp.int32),
            pltpu.VMEM((SCH,), jnp.int32),
            pltpu.VMEM((SCH,), jnp.int32),
            pltpu.VMEM((SCH, D), jnp.float32),
            pltpu.VMEM((SCH, D), jnp.float32),
            pltpu.VMEM((SCH, 128), jnp.float32),
            pltpu.SemaphoreType.DMA,
            pltpu.SemaphoreType.DMA,
            pltpu.SemaphoreType.DMA,
            pltpu.SemaphoreType.DMA,
        ],
    )
    def k(c_hbm, v_hbm, si_hbm, za_hbm, zd_hbm, aout, dout,
          acc_sh, den_sh, si_a, si_b, l8_v, cb_a, cb_b, vbuf,
          sem_sa, sem_sb, sem_ca, sem_cb):
        c = lax.axis_index("c")
        s = lax.axis_index("s")
        wid = c * NS + s
        a0 = s * APT
        d0 = s * DPT
        pltpu.sync_copy(zd_hbm.at[pl.ds(d0, DPT)], den_sh.at[pl.ds(d0, DPT)])
        eight = jnp.full((16,), 8, jnp.int32)
        nch = EPW // SCH

        for hp in range(H):
            col = pl.ds(hp * D, D)

            def start(j, sib, cb, sem_s, sem_c):
                e0 = wid * EPW + j * SCH
                pltpu.async_copy(si_hbm.at[pl.ds(e0, SCH)], sib, sem_s)
                pltpu.async_copy(c_hbm.at[pl.ds(e0, SCH), col], cb, sem_c)

            def wait(j, sib, cb, sem_s, sem_c):
                e0 = wid * EPW + j * SCH
                pltpu.make_async_copy(si_hbm.at[pl.ds(e0, SCH)], sib,
                                      sem_s).wait()
                pltpu.make_async_copy(c_hbm.at[pl.ds(e0, SCH), col], cb,
                                      sem_c).wait()

            def process(j, sib, cb, hp=hp):
                if hp == 0:
                    e0 = wid * EPW + j * SCH
                    pltpu.sync_copy(v_hbm.at[pl.ds(e0, SCH)], vbuf)
                    for kk in range(SCH // 16):
                        v = sib[pl.ds(kk * 16, 16)]
                        l8_v[pl.ds(kk * 16, 16)] = lax.div(v, eight)
                pltpu.sync_copy(cb, acc_sh.at[sib], add=True)
                if hp == 0:
                    pltpu.sync_copy(vbuf, den_sh.at[l8_v], add=True)

            pltpu.sync_copy(za_hbm.at[pl.ds(a0, APT)],
                            acc_sh.at[pl.ds(a0, APT)])
            plsc.subcore_barrier()
            start(0, si_a, cb_a, sem_sa, sem_ca)

            def body(g, carry):
                j0 = 2 * g
                start(j0 + 1, si_b, cb_b, sem_sb, sem_cb)
                wait(j0, si_a, cb_a, sem_sa, sem_ca)
                process(j0, si_a, cb_a)
                start(j0 + 2, si_a, cb_a, sem_sa, sem_ca)
                wait(j0 + 1, si_b, cb_b, sem_sb, sem_cb)
                process(j0 + 1, si_b, cb_b)
                return carry

            lax.fori_loop(0, (nch - 1) // 2, body, 0)
            wait(nch - 1, si_a, cb_a, sem_sa, sem_ca)
            process(nch - 1, si_a, cb_a)
            plsc.subcore_barrier()
            pltpu.sync_copy(acc_sh.at[pl.ds(a0, APT)],
                            aout.at[c, hp, pl.ds(a0, APT)])
        pltpu.sync_copy(den_sh.at[pl.ds(d0, DPT)], dout.at[c, pl.ds(d0, DPT)])

    return k(contrib, cvals, sidx, zacc, zden)


def _combine_tc(acc, den, x):
    NB = 1000

    def body(aref, dref, xref, oref):
        a = aref[...]
        d = dref[...][0] + dref[...][1]
        tot = xref[...]
        for hh in range(H):
            num = a[0, hh] + a[1, hh]
            tot = tot + (num / (d[:, hh:hh + 1] + 1e-10)) * (1.0 / H)
        oref[...] = tot

    return pl.pallas_call(
        body,
        grid=(N // NB,),
        in_specs=[
            pl.BlockSpec((NC, H, NB, D), lambda i: (0, 0, i, 0)),
            pl.BlockSpec((NC, NB, VW), lambda i: (0, i, 0)),
            pl.BlockSpec((NB, D), lambda i: (i, 0)),
        ],
        out_specs=pl.BlockSpec((NB, D), lambda i: (i, 0)),
        out_shape=jax.ShapeDtypeStruct((N, D), jnp.float32),
    )(acc, den, x)


def kernel(elem_weights, elem_in_fea, self_fea_idx, nbr_fea_idx,
           gate_w0, gate_b0, gate_w1, gate_b1,
           msg_w0, msg_b0, msg_w1, msg_b1, pow_p):
    w1d = elem_weights.reshape(N)
    sfea, nfea, wg = _gather_sc(elem_in_fea, w1d, self_fea_idx, nbr_fea_idx)

    w0s = jnp.concatenate([gate_w0[:, :, :D].reshape(H * HID, D),
                           msg_w0[:, :, :D].reshape(H * HID, D)],
                          axis=0).astype(jnp.bfloat16)
    w0n = jnp.concatenate([gate_w0[:, :, D:].reshape(H * HID, D),
                           msg_w0[:, :, D:].reshape(H * HID, D)],
                          axis=0).astype(jnp.bfloat16)
    b0 = jnp.concatenate([gate_b0.reshape(-1), msg_b0.reshape(-1)])
    b0r = jnp.broadcast_to(b0[None, :], (8, F1))
    w1g = jnp.zeros((H * HID, 8), jnp.float32)
    for hh in range(H):
        w1g = w1g.at[hh * HID:(hh + 1) * HID, hh].set(gate_w1[hh, 0])
    w1g = w1g.astype(jnp.bfloat16)
    mw1s = jnp.transpose(msg_w1, (0, 2, 1)).reshape(H * HID, D)
    mw1s = mw1s.astype(jnp.bfloat16)
    mb1p = jnp.concatenate([msg_b1, jnp.zeros((8 - H, D), jnp.float32)], axis=0)
    pow8 = jnp.concatenate([pow_p, jnp.zeros((8 - H,), jnp.float32)])
    b1g8 = jnp.concatenate([gate_b1[:, 0], jnp.zeros((8 - H,), jnp.float32)])
    misc = jnp.stack([pow8, b1g8] + [jnp.zeros((8,), jnp.float32)] * 6)

    w3 = wg.reshape(M // EB, 1, EB)
    si3 = self_fea_idx.reshape(M // EB, 1, EB)
    contrib, cvals = _dense_tc(sfea, nfea, w3, si3, w0s, w0n, b0r, w1g, mw1s,
                               mb1p, misc)

    zacc = jnp.zeros((NPAD, D), jnp.float32)
    zden = jnp.zeros((DNR, 128), jnp.float32)
    acc, denp = _scatter_sc(contrib, cvals, self_fea_idx, zacc, zden)
    den = denp.reshape(NC, NPAD, VW)

    return _combine_tc(acc, den, elem_in_fea)


# speedup vs baseline: 11.1089x; 1.0751x over previous
import functools

import jax
import jax.numpy as jnp
from jax import lax
from jax.experimental import pallas as pl
from jax.experimental.pallas import tpu as pltpu
from jax.experimental.pallas import tpu_sc as plsc

N = 10000
M = 320000
D = 128
H = 3
HID = 256

NC = 2
NS = 16
NW = NC * NS

EPW = M // NW
GCH = 80
SCH = 80
NPAD = 10240
APT = NPAD // NS
DNR = NPAD // 8
DPT = DNR // NS
CW = H * D
VW = 16
F1 = 2 * H * HID
EB = 1280


def _gather_sc(x, w, sidx, nidx):
    mesh = plsc.VectorSubcoreMesh(core_axis_name="c", subcore_axis_name="s")

    @functools.partial(
        pl.kernel,
        mesh=mesh,
        out_type=[
            jax.ShapeDtypeStruct((M, D), jnp.float32),
            jax.ShapeDtypeStruct((M, D), jnp.float32),
            jax.ShapeDtypeStruct((M,), jnp.float32),
        ],
        scratch_types=[
            pltpu.VMEM((GCH,), jnp.int32),
            pltpu.VMEM((GCH,), jnp.int32),
            pltpu.VMEM((GCH, D), jnp.float32),
            pltpu.VMEM((GCH, D), jnp.float32),
            pltpu.VMEM((GCH,), jnp.float32),
            pltpu.VMEM((GCH,), jnp.int32),
            pltpu.VMEM((GCH,), jnp.int32),
            pltpu.VMEM((GCH, D), jnp.float32),
            pltpu.VMEM((GCH, D), jnp.float32),
            pltpu.VMEM((GCH,), jnp.float32),
            pltpu.SemaphoreType.DMA,
            pltpu.SemaphoreType.DMA,
            pltpu.SemaphoreType.DMA,
            pltpu.SemaphoreType.DMA,
            pltpu.SemaphoreType.DMA,
            pltpu.SemaphoreType.DMA,
        ],
    )
    def k(x_hbm, w_hbm, si_hbm, ni_hbm, sout, nout, wout,
          si_a, ni_a, srow_a, nrow_a, wrow_a,
          si_b, ni_b, srow_b, nrow_b, wrow_b,
          sem_ia, sem_ga, sem_wa, sem_ib, sem_gb, sem_wb):
        wid = lax.axis_index("c") * NS + lax.axis_index("s")
        bufa = (si_a, ni_a, srow_a, nrow_a, wrow_a, sem_ia, sem_ga, sem_wa)
        bufb = (si_b, ni_b, srow_b, nrow_b, wrow_b, sem_ib, sem_gb, sem_wb)
        nch = EPW // GCH

        def idx_start(j, b):
            base = wid * EPW + j * GCH
            pltpu.async_copy(si_hbm.at[pl.ds(base, GCH)], b[0], b[5])
            pltpu.async_copy(ni_hbm.at[pl.ds(base, GCH)], b[1], b[5])

        def gathers_start(j, b):
            base = wid * EPW + j * GCH
            pltpu.make_async_copy(si_hbm.at[pl.ds(base, GCH)], b[0],
                                  b[5]).wait()
            pltpu.make_async_copy(ni_hbm.at[pl.ds(base, GCH)], b[1],
                                  b[5]).wait()
            pltpu.async_copy(x_hbm.at[b[0]], b[2], b[6])
            pltpu.async_copy(x_hbm.at[b[1]], b[3], b[6])
            pltpu.async_copy(w_hbm.at[b[1]], b[4], b[6])

        def gathers_wait(b):
            pltpu.make_async_copy(x_hbm.at[b[0]], b[2], b[6]).wait()
            pltpu.make_async_copy(x_hbm.at[b[1]], b[3], b[6]).wait()
            pltpu.make_async_copy(w_hbm.at[b[1]], b[4], b[6]).wait()

        def writes_start(j, b):
            base = wid * EPW + j * GCH
            pltpu.async_copy(b[2], sout.at[pl.ds(base, GCH)], b[7])
            pltpu.async_copy(b[3], nout.at[pl.ds(base, GCH)], b[7])
            pltpu.async_copy(b[4], wout.at[pl.ds(base, GCH)], b[7])

        def writes_wait(j, b):
            base = wid * EPW + j * GCH
            pltpu.make_async_copy(b[2], sout.at[pl.ds(base, GCH)],
                                  b[7]).wait()
            pltpu.make_async_copy(b[3], nout.at[pl.ds(base, GCH)],
                                  b[7]).wait()
            pltpu.make_async_copy(b[4], wout.at[pl.ds(base, GCH)],
                                  b[7]).wait()

        idx_start(0, bufa)
        gathers_start(0, bufa)

        def body(g, carry):
            j0 = 2 * g
            idx_start(j0 + 1, bufb)
            gathers_wait(bufa)
            writes_start(j0, bufa)
            gathers_start(j0 + 1, bufb)
            writes_wait(j0, bufa)
            idx_start(j0 + 2, bufa)
            gathers_wait(bufb)
            writes_start(j0 + 1, bufb)
            gathers_start(j0 + 2, bufa)
            writes_wait(j0 + 1, bufb)
            return carry

        lax.fori_loop(0, (nch - 1) // 2, body, 0)
        gathers_wait(bufa)
        base = wid * EPW + (nch - 1) * GCH
        pltpu.sync_copy(srow_a, sout.at[pl.ds(base, GCH)])
        pltpu.sync_copy(nrow_a, nout.at[pl.ds(base, GCH)])
        pltpu.sync_copy(wrow_a, wout.at[pl.ds(base, GCH)])

    return k(x, w, sidx, nidx)


def _dense_tc(sfea, nfea, w3, si3, w0s, w0n, b0r, w1g, mw1s, mb1p, misc):
    nblk = M // EB

    def body(sref, nref, wref, siref, w0s_r, w0n_r, b0_r, w1g_r, m1_r, mb1_r,
             mi_r, cout, vout):
        s = sref[...].astype(jnp.bfloat16)
        n = nref[...].astype(jnp.bfloat16)
        h = lax.dot_general(s, w0s_r[...], (((1,), (1,)), ((), ())),
                            preferred_element_type=jnp.float32)
        h = h + lax.dot_general(n, w0n_r[...], (((1,), (1,)), ((), ())),
                                preferred_element_type=jnp.float32)
        h = h + b0_r[...][0][None, :]
        h = jnp.where(h >= 0, h, 0.01 * h)
        hb = h.astype(jnp.bfloat16)
        g8 = lax.dot_general(hb[:, :H * HID], w1g_r[...],
                             (((1,), (0,)), ((), ())),
                             preferred_element_type=jnp.float32)
        g8 = g8 + mi_r[...][1][None, :]
        logw = jnp.log(wref[...][0, 0, :])
        c8 = jnp.exp(g8 + logw[:, None] * mi_r[...][0][None, :])
        outs = []
        for hh in range(H):
            hm = hb[:, H * HID + hh * HID:H * HID + (hh + 1) * HID]
            mh = lax.dot_general(hm, m1_r[...][hh * HID:(hh + 1) * HID, :],
                                 (((1,), (0,)), ((), ())),
                                 preferred_element_type=jnp.float32)
            mh = mh + mb1_r[...][hh][None, :]
            outs.append(c8[:, hh:hh + 1] * mh)
        cout[...] = jnp.concatenate(outs, axis=1)
        si = siref[...][0, 0, :]
        base = (lax.rem(si, jnp.full((EB,), 8, jnp.int32)) * 16)[:, None]
        colidx = lax.broadcasted_iota(jnp.int32, (EB, 128), 1)
        cpos = jnp.zeros((EB, 128), jnp.float32)
        for hh in range(H):
            cpos = cpos + jnp.where(colidx == base + hh,
                                    c8[:, hh:hh + 1], 0.0)
        vout[...] = cpos

    full = lambda shape: pl.BlockSpec(shape, lambda i: tuple(0 for _ in shape))
    return pl.pallas_call(
        body,
        grid=(nblk,),
        in_specs=[
            pl.BlockSpec((EB, D), lambda i: (i, 0)),
            pl.BlockSpec((EB, D), lambda i: (i, 0)),
            pl.BlockSpec((1, 1, EB), lambda i: (i, 0, 0)),
            pl.BlockSpec((1, 1, EB), lambda i: (i, 0, 0)),
            full((F1, D)),
            full((F1, D)),
            full((8, F1)),
            full((H * HID, 8)),
            full((H * HID, D)),
            full((8, D)),
            full((8, 8)),
        ],
        out_specs=[
            pl.BlockSpec((EB, CW), lambda i: (i, 0)),
            pl.BlockSpec((EB, 128), lambda i: (i, 0)),
        ],
        out_shape=[
            jax.ShapeDtypeStruct((M, CW), jnp.float32),
            jax.ShapeDtypeStruct((M, 128), jnp.float32),
        ],
    )(sfea, nfea, w3, si3, w0s, w0n, b0r, w1g, mw1s, mb1p, misc)


def _scatter_sc(contrib, cvals, sidx, zacc, zden):
    mesh = plsc.VectorSubcoreMesh(core_axis_name="c", subcore_axis_name="s")

    @functools.partial(
        pl.kernel,
        mesh=mesh,
        out_type=[
            jax.ShapeDtypeStruct((NC, H, NPAD, D), jnp.float32),
            jax.ShapeDtypeStruct((NC, DNR, 128), jnp.float32),
        ],
        scratch_types=[
            pltpu.VMEM_SHARED((NPAD, D), jnp.float32),
            pltpu.VMEM_SHARED((DNR, 128), jnp.float32),
            pltpu.VMEM((SCH,), jnp.int32),
            pltpu.VMEM((SCH,), jnp.int32),
            pltpu.VMEM((SCH,), jnp.int32),
            pltpu.VMEM((SCH, D), jnp.float32),
            pltpu.VMEM((SCH, D), jnp.float32),
            pltpu.VMEM((SCH, 128), jnp.float32),
            pltpu.SemaphoreType.DMA,
            pltpu.SemaphoreType.DMA,
            pltpu.SemaphoreType.DMA,
            pltpu.SemaphoreType.DMA,
        ],
    )
    def k(c_hbm, v_hbm, si_hbm, za_hbm, zd_hbm, aout, dout,
          acc_sh, den_sh, si_a, si_b, l8_v, cb_a, cb_b, vbuf,
          sem_sa, sem_sb, sem_ca, sem_cb):
        c = lax.axis_index("c")
        s = lax.axis_index("s")
        wid = c * NS + s
        a0 = s * APT
        d0 = s * DPT
        pltpu.sync_copy(zd_hbm.at[pl.ds(d0, DPT)], den_sh.at[pl.ds(d0, DPT)])
        eight = jnp.full((16,), 8, jnp.int32)
        nch = EPW // SCH

        for hp in range(H):
            col = pl.ds(hp * D, D)

            def start(j, sib, cb, sem_s, sem_c):
                e0 = wid * EPW + j * SCH
                pltpu.async_copy(si_hbm.at[pl.ds(e0, SCH)], sib, sem_s)
                pltpu.async_copy(c_hbm.at[pl.ds(e0, SCH), col], cb, sem_c)

            def wait(j, sib, cb, sem_s, sem_c):
                e0 = wid * EPW + j * SCH
                pltpu.make_async_copy(si_hbm.at[pl.ds(e0, SCH)], sib,
                                      sem_s).wait()
                pltpu.make_async_copy(c_hbm.at[pl.ds(e0, SCH), col], cb,
                                      sem_c).wait()

            def process(j, sib, cb, hp=hp):
                if hp == 0:
                    e0 = wid * EPW + j * SCH
                    pltpu.sync_copy(v_hbm.at[pl.ds(e0, SCH)], vbuf)
                    for kk in range(SCH // 16):
                        v = sib[pl.ds(kk * 16, 16)]
                        l8_v[pl.ds(kk * 16, 16)] = lax.div(v, eight)
                pltpu.sync_copy(cb, acc_sh.at[sib], add=True)
                if hp == 0:
                    pltpu.sync_copy(vbuf, den_sh.at[l8_v], add=True)

            pltpu.sync_copy(za_hbm.at[pl.ds(a0, APT)],
                            acc_sh.at[pl.ds(a0, APT)])
            plsc.subcore_barrier()
            start(0, si_a, cb_a, sem_sa, sem_ca)

            def body(g, carry):
                j0 = 2 * g
                start(j0 + 1, si_b, cb_b, sem_sb, sem_cb)
                wait(j0, si_a, cb_a, sem_sa, sem_ca)
                process(j0, si_a, cb_a)
                start(j0 + 2, si_a, cb_a, sem_sa, sem_ca)
                wait(j0 + 1, si_b, cb_b, sem_sb, sem_cb)
                process(j0 + 1, si_b, cb_b)
                return carry

            lax.fori_loop(0, (nch - 1) // 2, body, 0)
            wait(nch - 1, si_a, cb_a, sem_sa, sem_ca)
            process(nch - 1, si_a, cb_a)
            plsc.subcore_barrier()
            pltpu.sync_copy(acc_sh.at[pl.ds(a0, APT)],
                            aout.at[c, hp, pl.ds(a0, APT)])
        pltpu.sync_copy(den_sh.at[pl.ds(d0, DPT)], dout.at[c, pl.ds(d0, DPT)])

    return k(contrib, cvals, sidx, zacc, zden)


def _combine_tc(acc, den, x):
    NB = 1000

    def body(aref, dref, xref, oref):
        a = aref[...]
        d = dref[...][0] + dref[...][1]
        tot = xref[...]
        for hh in range(H):
            num = a[0, hh] + a[1, hh]
            tot = tot + (num / (d[:, hh:hh + 1] + 1e-10)) * (1.0 / H)
        oref[...] = tot

    return pl.pallas_call(
        body,
        grid=(N // NB,),
        in_specs=[
            pl.BlockSpec((NC, H, NB, D), lambda i: (0, 0, i, 0)),
            pl.BlockSpec((NC, NB, VW), lambda i: (0, i, 0)),
            pl.BlockSpec((NB, D), lambda i: (i, 0)),
        ],
        out_specs=pl.BlockSpec((NB, D), lambda i: (i, 0)),
        out_shape=jax.ShapeDtypeStruct((N, D), jnp.float32),
    )(acc, den, x)


def kernel(elem_weights, elem_in_fea, self_fea_idx, nbr_fea_idx,
           gate_w0, gate_b0, gate_w1, gate_b1,
           msg_w0, msg_b0, msg_w1, msg_b1, pow_p):
    w1d = elem_weights.reshape(N)
    sfea, nfea, wg = _gather_sc(elem_in_fea, w1d, self_fea_idx, nbr_fea_idx)

    w0s = jnp.concatenate([gate_w0[:, :, :D].reshape(H * HID, D),
                           msg_w0[:, :, :D].reshape(H * HID, D)],
                          axis=0).astype(jnp.bfloat16)
    w0n = jnp.concatenate([gate_w0[:, :, D:].reshape(H * HID, D),
                           msg_w0[:, :, D:].reshape(H * HID, D)],
                          axis=0).astype(jnp.bfloat16)
    b0 = jnp.concatenate([gate_b0.reshape(-1), msg_b0.reshape(-1)])
    b0r = jnp.broadcast_to(b0[None, :], (8, F1))
    w1g = jnp.zeros((H * HID, 8), jnp.float32)
    for hh in range(H):
        w1g = w1g.at[hh * HID:(hh + 1) * HID, hh].set(gate_w1[hh, 0])
    w1g = w1g.astype(jnp.bfloat16)
    mw1s = jnp.transpose(msg_w1, (0, 2, 1)).reshape(H * HID, D)
    mw1s = mw1s.astype(jnp.bfloat16)
    mb1p = jnp.concatenate([msg_b1, jnp.zeros((8 - H, D), jnp.float32)], axis=0)
    pow8 = jnp.concatenate([pow_p, jnp.zeros((8 - H,), jnp.float32)])
    b1g8 = jnp.concatenate([gate_b1[:, 0], jnp.zeros((8 - H,), jnp.float32)])
    misc = jnp.stack([pow8, b1g8] + [jnp.zeros((8,), jnp.float32)] * 6)

    w3 = wg.reshape(M // EB, 1, EB)
    si3 = self_fea_idx.reshape(M // EB, 1, EB)
    contrib, cvals = _dense_tc(sfea, nfea, w3, si3, w0s, w0n, b0r, w1g, mw1s,
                               mb1p, misc)

    zacc = jnp.zeros((NPAD, D), jnp.float32)
    zden = jnp.zeros((DNR, 128), jnp.float32)
    acc, denp = _scatter_sc(contrib, cvals, self_fea_idx, zacc, zden)
    den = denp.reshape(NC, NPAD, VW)

    return _combine_tc(acc, den, elem_in_fea)
